# baseline (speedup 1.0000x reference)
import functools

import jax
import jax.numpy as jnp
from jax import lax
from jax.experimental import pallas as pl
from jax.experimental.pallas import tpu as pltpu
from jax.experimental.pallas import tpu_sc as plsc

DIM = 128
BATCH = 16384

LANES = 16
NUM_WORKERS = 32
B_PER_W = BATCH // NUM_WORKERS
CHUNK = 128
N_CHUNKS = B_PER_W // CHUNK
NBUF = 3
GROUP = 2 * CHUNK

GROUPS = ((0, 0, 2), (0, 2, 2),
          (1, 0, 2), (1, 2, 2),
          (2, 0, 2), (2, 2, 2))


def _mf_body(user_table, item_table, user_list, pos_items, neg_items,
             user_out, pos_out, neg_out, partials,
             idx_all, bufs, acc_v, isem, *sems):
    gsem = sems[:NBUF]
    wsem = sems[NBUF:]
    tables = (user_table, item_table, item_table)
    idxs = (user_list, pos_items, neg_items)
    outs = (user_out, pos_out, neg_out)

    nc = 2
    wid = lax.axis_index("s") * nc + lax.axis_index("c")
    base = wid * B_PER_W

    crow = wid * N_CHUNKS
    idescs = [
        pltpu.async_copy(idx_hbm.at[pl.ds(crow, N_CHUNKS)],
                         idx_all.at[pl.ds(t * N_CHUNKS, N_CHUNKS)], isem)
        for t, idx_hbm in enumerate(idxs)
    ]
    idx_waited = [False] * 3

    def fire(grp):
        t, c0, nch = GROUPS[grp]
        if not idx_waited[t]:
            idescs[t].wait()
            idx_waited[t] = True
        b = grp % NBUF
        return [
            pltpu.async_copy(
                tables[t].at[idx_all.at[t * N_CHUNKS + c0 + k2]],
                bufs.at[b, pl.ds(k2 * CHUNK, CHUNK)], gsem[b])
            for k2 in range(nch)
        ]

    gdescs = [None] * len(GROUPS)
    for grp in range(NBUF):
        gdescs[grp] = fire(grp)

    accs = tuple(jnp.zeros((LANES,), jnp.float32) for _ in range(4))

    for grp in range(len(GROUPS)):
        t, c0, nch = GROUPS[grp]
        b = grp % NBUF
        rows = nch * CHUNK
        off = base + c0 * CHUNK
        for d in gdescs[grp]:
            d.wait()
        wdesc = pltpu.async_copy(
            bufs.at[b, pl.ds(0, rows)], outs[t].at[pl.ds(off, rows)],
            wsem[b])

        def body(r, xs, b=b):
            xs = list(xs)
            for rr in range(2):
                for cc in range(8):
                    v = bufs[b, 2 * r + rr, pl.ds(cc * LANES, LANES)]
                    xs[cc % 4] = xs[cc % 4] + v * v
            return tuple(xs)

        accs = lax.fori_loop(0, rows // 2, body, accs)
        wdesc.wait()
        if grp + NBUF < len(GROUPS):
            gdescs[grp + NBUF] = fire(grp + NBUF)

    acc_v[...] = (accs[0] + accs[1]) + (accs[2] + accs[3])
    pltpu.sync_copy(acc_v, partials.at[wid])


@jax.jit
def kernel(user_table, item_table, user_list, pos_items, neg_items):
    mesh = plsc.VectorSubcoreMesh(core_axis_name="c", subcore_axis_name="s")
    f = functools.partial(
        pl.kernel,
        mesh=mesh,
        out_type=[
            jax.ShapeDtypeStruct((BATCH, DIM), jnp.float32),
            jax.ShapeDtypeStruct((BATCH, DIM), jnp.float32),
            jax.ShapeDtypeStruct((BATCH, DIM), jnp.float32),
            jax.ShapeDtypeStruct((NUM_WORKERS, LANES), jnp.float32),
        ],
        scratch_types=[
            pltpu.VMEM((3 * N_CHUNKS, CHUNK), jnp.int32),
            pltpu.VMEM((NBUF, GROUP, DIM), jnp.float32),
            pltpu.VMEM((LANES,), jnp.float32),
        ] + [pltpu.SemaphoreType.DMA] * (1 + 2 * NBUF),
    )(_mf_body)
    user_emb, posI_emb, negI_emb, partials = f(
        user_table, item_table,
        user_list.astype(jnp.int32).reshape(BATCH // CHUNK, CHUNK),
        pos_items.astype(jnp.int32).reshape(BATCH // CHUNK, CHUNK),
        neg_items.astype(jnp.int32).reshape(BATCH // CHUNK, CHUNK),
    )
    reg = jnp.sum(partials) / jnp.float32(BATCH)
    return (user_emb, posI_emb, negI_emb, reg)
